# Initial kernel scaffold; baseline (speedup 1.0000x reference)
import jax
import jax.numpy as jnp
from jax.experimental import pallas as pl


def kernel(x, edge_index, W1, b1, W2, b2, Wlin, blin):
    raise NotImplementedError("write your pallas kernel here")


# speedup vs baseline: 16.0297x; 16.0297x over previous
import functools

import jax
import jax.numpy as jnp
from jax import lax
from jax.experimental import pallas as pl
from jax.experimental.pallas import tpu as pltpu
from jax.experimental.pallas import tpu_sc as plsc

N = 10000
NPAD = 10240
E = 320000
IN_CH = 128
D = 64

NC = 2
NS = 16
NW = NC * NS
CH = 128
EW = ((E + NW * CH - 1) // (NW * CH)) * CH
EPAD = EW * NW
NCHUNK = EW // CH
RPS = NPAD // NS

_mesh = plsc.VectorSubcoreMesh(
    core_axis_name="c", subcore_axis_name="s", num_cores=NC, num_subcores=NS
)


def _zero_rows(zbuf, width_vregs, nrows):
    zero = jnp.zeros((16,), jnp.float32)

    def body(i, _):
        for j in range(width_vregs):
            zbuf[i, pl.ds(j * 16, 16)] = zero
        return 0

    lax.fori_loop(0, nrows, body, 0, unroll=4)


def _sc_deg_body(dst_hbm, out, onesb, zbuf, dstv, acc_sh):
    c = lax.axis_index("c")
    s = lax.axis_index("s")
    wid = c * NS + s

    one = jnp.ones((16,), jnp.float32)

    def fill_ones(i, _):
        onesb[i, pl.ds(0, 16)] = one
        return 0

    lax.fori_loop(0, CH, fill_ones, 0, unroll=4)
    _zero_rows(zbuf, 1, 64)
    for r in range(RPS // 64):
        pltpu.sync_copy(zbuf, acc_sh.at[pl.ds(s * RPS + r * 64, 64)])
    plsc.subcore_barrier()

    wbase = wid * EW

    def chunk(i, _):
        pltpu.sync_copy(dst_hbm.at[pl.ds(wbase + i * CH, CH)], dstv)
        pltpu.sync_copy(onesb, acc_sh.at[dstv], add=True)
        return 0

    lax.fori_loop(0, NCHUNK, chunk, 0)
    plsc.subcore_barrier()
    pltpu.sync_copy(acc_sh.at[pl.ds(s * RPS, RPS)], out.at[c, pl.ds(s * RPS, RPS)])


_sc_deg = pl.kernel(
    _sc_deg_body,
    out_type=jax.ShapeDtypeStruct((NC, NPAD, 16), jnp.float32),
    mesh=_mesh,
    scratch_types=[
        pltpu.VMEM((CH, 16), jnp.float32),
        pltpu.VMEM((64, 16), jnp.float32),
        pltpu.VMEM((CH,), jnp.int32),
        pltpu.VMEM_SHARED((NPAD, 16), jnp.float32),
    ],
    compiler_params=pltpu.CompilerParams(use_tc_tiling_on_sc=False),
)


def _sc_scatter_body(g_hbm, src_hbm, dst_hbm, out0, out1,
                     srcv, dstv, rows, zbuf, acc_sh, sem):
    c = lax.axis_index("c")
    s = lax.axis_index("s")
    wid = c * NS + s

    _zero_rows(zbuf, 4, 64)
    for r in range(RPS // 64):
        pltpu.sync_copy(zbuf, acc_sh.at[pl.ds(s * RPS + r * 64, 64)])
    plsc.subcore_barrier()

    wbase = wid * EW

    def chunk(i, _):
        base = wbase + i * CH
        pltpu.sync_copy(src_hbm.at[pl.ds(base, CH)], srcv)
        pltpu.async_copy(g_hbm.at[srcv], rows, sem).wait()
        pltpu.sync_copy(dst_hbm.at[pl.ds(base, CH)], dstv)
        pltpu.sync_copy(rows, acc_sh.at[dstv], add=True)
        return 0

    lax.fori_loop(0, NCHUNK, chunk, 0)
    plsc.subcore_barrier()

    @pl.when(c == 0)
    def _():
        pltpu.sync_copy(acc_sh.at[pl.ds(s * RPS, RPS)], out0.at[pl.ds(s * RPS, RPS)])

    @pl.when(c == 1)
    def _():
        pltpu.sync_copy(acc_sh.at[pl.ds(s * RPS, RPS)], out1.at[pl.ds(s * RPS, RPS)])


_sc_scatter = pl.kernel(
    _sc_scatter_body,
    out_type=(
        jax.ShapeDtypeStruct((NPAD, D), jnp.float32),
        jax.ShapeDtypeStruct((NPAD, D), jnp.float32),
    ),
    mesh=_mesh,
    scratch_types=[
        pltpu.VMEM((CH,), jnp.int32),
        pltpu.VMEM((CH,), jnp.int32),
        pltpu.VMEM((CH, D), jnp.float32),
        pltpu.VMEM((64, D), jnp.float32),
        pltpu.VMEM_SHARED((NPAD, D), jnp.float32),
        pltpu.SemaphoreType.DMA,
    ],
    compiler_params=pltpu.CompilerParams(use_tc_tiling_on_sc=False),
)


TCR = 2048
TCG = NPAD // TCR


def _dinv(d0r, d1r):
    deg = d0r[:, :1] + d1r[:, :1] + 1.0
    return lax.rsqrt(jnp.maximum(deg, 1.0))


def _tc1_body(xr, wr, d0r, d1r, h1r, g1r):
    dinv = _dinv(d0r, d1r)
    h = jnp.dot(xr[:], wr[:], preferred_element_type=jnp.float32,
                precision=lax.Precision.HIGHEST)
    h1r[:] = h
    g1r[:] = h * dinv


def _tc2_body(p0r, p1r, d0r, d1r, h1r, b1r, w2r, h2r, g2r):
    i = pl.program_id(0)
    dinv = _dinv(d0r, d1r)
    z = jnp.maximum(dinv * (p0r[:] + p1r[:]) + (dinv * dinv) * h1r[:] + b1r[:], 0.0)
    rows = i * TCR + lax.broadcasted_iota(jnp.int32, (TCR, 1), 0)
    z = jnp.where(rows < N, z, 0.0)
    h2 = jnp.dot(z, w2r[:], preferred_element_type=jnp.float32,
                 precision=lax.Precision.HIGHEST)
    h2r[:] = h2
    g2r[:] = h2 * dinv


def _tc3_body(p0r, p1r, d0r, d1r, h2r, b2r, wlr, blr, outr):
    dinv = _dinv(d0r, d1r)
    z = jnp.maximum(dinv * (p0r[:] + p1r[:]) + (dinv * dinv) * h2r[:] + b2r[:], 0.0)
    outr[:] = jnp.sum(z * wlr[:], axis=1, keepdims=True) + blr[:]


def _row_spec(w):
    return pl.BlockSpec((TCR, w), lambda i: (i, 0))


def _full_spec(a, b):
    return pl.BlockSpec((a, b), lambda i: (0, 0))


_tc1 = pl.pallas_call(
    _tc1_body,
    grid=(TCG,),
    in_specs=[_row_spec(IN_CH), _full_spec(IN_CH, D), _row_spec(16), _row_spec(16)],
    out_specs=[_row_spec(D), _row_spec(D)],
    out_shape=[
        jax.ShapeDtypeStruct((NPAD, D), jnp.float32),
        jax.ShapeDtypeStruct((NPAD, D), jnp.float32),
    ],
)

_tc2 = pl.pallas_call(
    _tc2_body,
    grid=(TCG,),
    in_specs=[_row_spec(D), _row_spec(D), _row_spec(16), _row_spec(16),
              _row_spec(D), _full_spec(1, D), _full_spec(D, D)],
    out_specs=[_row_spec(D), _row_spec(D)],
    out_shape=[
        jax.ShapeDtypeStruct((NPAD, D), jnp.float32),
        jax.ShapeDtypeStruct((NPAD, D), jnp.float32),
    ],
)

_tc3 = pl.pallas_call(
    _tc3_body,
    grid=(TCG,),
    in_specs=[_row_spec(D), _row_spec(D), _row_spec(16), _row_spec(16),
              _row_spec(D), _full_spec(1, D), _full_spec(1, D), _full_spec(1, 1)],
    out_specs=[pl.BlockSpec((TCR, 1), lambda i: (i, 0))],
    out_shape=[jax.ShapeDtypeStruct((NPAD, 1), jnp.float32)],
)


@jax.jit
def kernel(x, edge_index, W1, b1, W2, b2, Wlin, blin):
    ei = edge_index.astype(jnp.int32)
    pad = EPAD - E
    src = jnp.concatenate([ei[0], jnp.full((pad,), N, jnp.int32)])
    dst = jnp.concatenate([ei[1], jnp.full((pad,), N, jnp.int32)])
    x_pad = jnp.pad(x, ((0, NPAD - N), (0, 0)))
    b1r = b1.reshape(1, D)
    b2r = b2.reshape(1, D)
    wlr = Wlin.reshape(1, D)
    blr = blin.reshape(1, 1)

    dd = _sc_deg(dst)
    d0, d1 = dd[0], dd[1]
    h1, g1 = _tc1(x_pad, W1, d0, d1)
    p0, p1 = _sc_scatter(g1, src, dst)
    h2, g2 = _tc2(p0, p1, d0, d1, h1, b1r, W2)
    q0, q1 = _sc_scatter(g2, src, dst)
    out = _tc3(q0, q1, d0, d1, h2, b2r, wlr, blr)[0]
    return out[:N, 0]


# speedup vs baseline: 19.2291x; 1.1996x over previous
import functools

import jax
import jax.numpy as jnp
from jax import lax
from jax.experimental import pallas as pl
from jax.experimental.pallas import tpu as pltpu
from jax.experimental.pallas import tpu_sc as plsc

N = 10000
NPAD = 10240
E = 320000
IN_CH = 128
D = 64

NC = 2
NS = 16
NW = NC * NS
CH = 128
NBUF = 4
NCHUNK = 80
NGRP = NCHUNK // NBUF
EW = NCHUNK * CH
EPAD = EW * NW
TOTCH = EPAD // CH
RPS = NPAD // NS

_mesh = plsc.VectorSubcoreMesh(
    core_axis_name="c", subcore_axis_name="s", num_cores=NC, num_subcores=NS
)
_sc_params = pltpu.CompilerParams(use_tc_tiling_on_sc=False)


def _zero_rows(zbuf, width_vregs, nrows):
    zero = jnp.zeros((16,), jnp.float32)

    def body(i, _):
        for j in range(width_vregs):
            zbuf[i, pl.ds(j * 16, 16)] = zero
        return 0

    lax.fori_loop(0, nrows, body, 0, unroll=4)


def _zero_acc(zbuf, acc_sh, s, width):
    _zero_rows(zbuf, width // 16, 64)
    for r in range(RPS // 64):
        pltpu.sync_copy(zbuf, acc_sh.at[pl.ds(s * RPS + r * 64, 64)])
    plsc.subcore_barrier()


def _sc_deg_body(dst_hbm, out, onesb, zbuf, didx, acc_sh, isem):
    c = lax.axis_index("c")
    s = lax.axis_index("s")
    wid = c * NS + s
    wch = wid * NCHUNK

    one = jnp.ones((16,), jnp.float32)

    def fill_ones(i, _):
        onesb[i, pl.ds(0, 16)] = one
        return 0

    lax.fori_loop(0, CH, fill_ones, 0, unroll=4)
    _zero_acc(zbuf, acc_sh, s, 16)

    def stage(gi, p):
        pltpu.async_copy(dst_hbm.at[pl.ds(wch + gi * NBUF, NBUF)], didx.at[p],
                         isem.at[p])

    def wait_stage(p):
        pltpu.make_async_copy(dst_hbm.at[pl.ds(0, NBUF)], didx.at[p],
                              isem.at[p]).wait()

    def scatters(p):
        for b in range(NBUF):
            pltpu.sync_copy(onesb, acc_sh.at[didx.at[p, b]], add=True)

    stage(0, 0)
    stage(1, 1)

    def go_body(go, _):
        g0 = 2 * go
        wait_stage(0)
        scatters(0)

        @pl.when(g0 + 2 < NGRP)
        def _():
            stage(g0 + 2, 0)

        wait_stage(1)
        scatters(1)

        @pl.when(g0 + 3 < NGRP)
        def _():
            stage(g0 + 3, 1)

        return 0

    lax.fori_loop(0, NGRP // 2, go_body, 0)
    plsc.subcore_barrier()
    pltpu.sync_copy(acc_sh.at[pl.ds(s * RPS, RPS)], out.at[c, pl.ds(s * RPS, RPS)])


_sc_deg = pl.kernel(
    _sc_deg_body,
    out_type=jax.ShapeDtypeStruct((NC, NPAD, 16), jnp.float32),
    mesh=_mesh,
    scratch_types=[
        pltpu.VMEM((CH, 16), jnp.float32),
        pltpu.VMEM((64, 16), jnp.float32),
        pltpu.VMEM((2, NBUF, CH), jnp.int32),
        pltpu.VMEM_SHARED((NPAD, 16), jnp.float32),
        pltpu.SemaphoreType.DMA((2,)),
    ],
    compiler_params=_sc_params,
)


def _sc_scatter_body(g_hbm, src_hbm, dst_hbm, out,
                     sidx, didx, rows, zbuf, acc_sh, ssem, dsem, gsem):
    c = lax.axis_index("c")
    s = lax.axis_index("s")
    wid = c * NS + s
    wch = wid * NCHUNK

    _zero_acc(zbuf, acc_sh, s, D)

    def stage_src(gi, p):
        pltpu.async_copy(src_hbm.at[pl.ds(wch + gi * NBUF, NBUF)], sidx.at[p],
                         ssem.at[p])

    def stage_dst(gi, p):
        pltpu.async_copy(dst_hbm.at[pl.ds(wch + gi * NBUF, NBUF)], didx.at[p],
                         dsem.at[p])

    def wait_src(p):
        pltpu.make_async_copy(src_hbm.at[pl.ds(0, NBUF)], sidx.at[p],
                              ssem.at[p]).wait()

    def wait_dst(p):
        pltpu.make_async_copy(dst_hbm.at[pl.ds(0, NBUF)], didx.at[p],
                              dsem.at[p]).wait()

    def fire(p):
        for b in range(NBUF):
            pltpu.async_copy(g_hbm.at[sidx.at[p, b]], rows.at[p, b],
                             gsem.at[p, b])

    def wait_gathers(p):
        for b in range(NBUF):
            pltpu.make_async_copy(g_hbm.at[sidx.at[p, b]], rows.at[p, b],
                                  gsem.at[p, b]).wait()

    def scatters(p):
        for b in range(NBUF):
            pltpu.sync_copy(rows.at[p, b], acc_sh.at[didx.at[p, b]], add=True)

    stage_src(0, 0)
    stage_dst(0, 0)
    wait_src(0)
    fire(0)
    stage_src(1, 1)
    stage_dst(1, 1)

    def go_body(go, _):
        g0 = 2 * go
        wait_src(1)
        fire(1)
        wait_gathers(0)

        @pl.when(g0 + 2 < NGRP)
        def _():
            stage_src(g0 + 2, 0)

        wait_dst(0)
        scatters(0)

        @pl.when(g0 + 2 < NGRP)
        def _():
            stage_dst(g0 + 2, 0)
            wait_src(0)
            fire(0)

        wait_gathers(1)

        @pl.when(g0 + 3 < NGRP)
        def _():
            stage_src(g0 + 3, 1)

        wait_dst(1)
        scatters(1)

        @pl.when(g0 + 3 < NGRP)
        def _():
            stage_dst(g0 + 3, 1)

        return 0

    lax.fori_loop(0, NGRP // 2, go_body, 0)
    plsc.subcore_barrier()
    pltpu.sync_copy(acc_sh.at[pl.ds(s * RPS, RPS)], out.at[c, pl.ds(s * RPS, RPS)])


_sc_scatter = pl.kernel(
    _sc_scatter_body,
    out_type=jax.ShapeDtypeStruct((NC, NPAD, D), jnp.float32),
    mesh=_mesh,
    scratch_types=[
        pltpu.VMEM((2, NBUF, CH), jnp.int32),
        pltpu.VMEM((2, NBUF, CH), jnp.int32),
        pltpu.VMEM((2, NBUF, CH, D), jnp.float32),
        pltpu.VMEM((64, D), jnp.float32),
        pltpu.VMEM_SHARED((NPAD, D), jnp.float32),
        pltpu.SemaphoreType.DMA((2,)),
        pltpu.SemaphoreType.DMA((2,)),
        pltpu.SemaphoreType.DMA((2, NBUF)),
    ],
    compiler_params=_sc_params,
)


TCR = 2048
TCG = NPAD // TCR


def _dinv(d0r, d1r):
    deg = d0r[:, :1] + d1r[:, :1] + 1.0
    return lax.rsqrt(jnp.maximum(deg, 1.0))


def _tc1_body(xr, wr, d0r, d1r, h1r, g1r):
    dinv = _dinv(d0r, d1r)
    h = jnp.dot(xr[:], wr[:], preferred_element_type=jnp.float32,
                precision=lax.Precision.HIGHEST)
    h1r[:] = h
    g1r[:] = h * dinv


def _tc2_body(p0r, p1r, d0r, d1r, h1r, b1r, w2r, h2r, g2r):
    i = pl.program_id(0)
    dinv = _dinv(d0r, d1r)
    z = jnp.maximum(dinv * (p0r[:] + p1r[:]) + (dinv * dinv) * h1r[:] + b1r[:], 0.0)
    rows = i * TCR + lax.broadcasted_iota(jnp.int32, (TCR, 1), 0)
    z = jnp.where(rows < N, z, 0.0)
    h2 = jnp.dot(z, w2r[:], preferred_element_type=jnp.float32,
                 precision=lax.Precision.HIGHEST)
    h2r[:] = h2
    g2r[:] = h2 * dinv


def _tc3_body(p0r, p1r, d0r, d1r, h2r, b2r, wlr, blr, outr):
    dinv = _dinv(d0r, d1r)
    z = jnp.maximum(dinv * (p0r[:] + p1r[:]) + (dinv * dinv) * h2r[:] + b2r[:], 0.0)
    outr[:] = jnp.sum(z * wlr[:], axis=1, keepdims=True) + blr[:]


def _row_spec(w):
    return pl.BlockSpec((TCR, w), lambda i: (i, 0))


def _full_spec(a, b):
    return pl.BlockSpec((a, b), lambda i: (0, 0))


_tc1 = pl.pallas_call(
    _tc1_body,
    grid=(TCG,),
    in_specs=[_row_spec(IN_CH), _full_spec(IN_CH, D), _row_spec(16), _row_spec(16)],
    out_specs=[_row_spec(D), _row_spec(D)],
    out_shape=[
        jax.ShapeDtypeStruct((NPAD, D), jnp.float32),
        jax.ShapeDtypeStruct((NPAD, D), jnp.float32),
    ],
)

_tc2 = pl.pallas_call(
    _tc2_body,
    grid=(TCG,),
    in_specs=[_row_spec(D), _row_spec(D), _row_spec(16), _row_spec(16),
              _row_spec(D), _full_spec(1, D), _full_spec(D, D)],
    out_specs=[_row_spec(D), _row_spec(D)],
    out_shape=[
        jax.ShapeDtypeStruct((NPAD, D), jnp.float32),
        jax.ShapeDtypeStruct((NPAD, D), jnp.float32),
    ],
)

_tc3 = pl.pallas_call(
    _tc3_body,
    grid=(TCG,),
    in_specs=[_row_spec(D), _row_spec(D), _row_spec(16), _row_spec(16),
              _row_spec(D), _full_spec(1, D), _full_spec(1, D), _full_spec(1, 1)],
    out_specs=[pl.BlockSpec((TCR, 1), lambda i: (i, 0))],
    out_shape=[jax.ShapeDtypeStruct((NPAD, 1), jnp.float32)],
)


@jax.jit
def kernel(x, edge_index, W1, b1, W2, b2, Wlin, blin):
    ei = edge_index.astype(jnp.int32)
    pad = EPAD - E
    trash = N + jnp.arange(pad, dtype=jnp.int32) % (NPAD - N)
    src = jnp.concatenate([ei[0], jnp.full((pad,), N, jnp.int32)]).reshape(TOTCH, CH)
    dst = jnp.concatenate([ei[1], trash]).reshape(TOTCH, CH)
    x_pad = jnp.pad(x, ((0, NPAD - N), (0, 0)))
    b1r = b1.reshape(1, D)
    b2r = b2.reshape(1, D)
    wlr = Wlin.reshape(1, D)
    blr = blin.reshape(1, 1)

    dd = _sc_deg(dst)
    d0, d1 = dd[0], dd[1]
    h1, g1 = _tc1(x_pad, W1, d0, d1)
    pp = _sc_scatter(g1, src, dst)
    h2, g2 = _tc2(pp[0], pp[1], d0, d1, h1, b1r, W2)
    qq = _sc_scatter(g2, src, dst)
    out = _tc3(qq[0], qq[1], d0, d1, h2, b2r, wlr, blr)[0]
    return out[:N, 0]


# speedup vs baseline: 32.2465x; 1.6770x over previous
import functools

import jax
import jax.numpy as jnp
from jax import lax
from jax.experimental import pallas as pl
from jax.experimental.pallas import tpu as pltpu
from jax.experimental.pallas import tpu_sc as plsc

N = 10000
NPAD = 10240
E = 320000
IN_CH = 128
D = 64
DH = D // 2

NC = 2
NS = 16
NW = NC * NS
CH = 128
NBUF = 4
TOTCH = 2560
EPAD = TOTCH * CH
RPS = NPAD // NS

DEG_NCHUNK = TOTCH // NW
DEG_NGRP = DEG_NCHUNK // NBUF
SCT_NCHUNK = TOTCH // NS
SCT_NGRP = SCT_NCHUNK // NBUF

_mesh = plsc.VectorSubcoreMesh(
    core_axis_name="c", subcore_axis_name="s", num_cores=NC, num_subcores=NS
)
_sc_params = pltpu.CompilerParams(use_tc_tiling_on_sc=False)


def _zero_rows(zbuf, width_vregs, nrows):
    zero = jnp.zeros((16,), jnp.float32)

    def body(i, _):
        for j in range(width_vregs):
            zbuf[i, pl.ds(j * 16, 16)] = zero
        return 0

    lax.fori_loop(0, nrows, body, 0, unroll=4)


def _zero_acc(zbuf, acc_sh, s, width):
    _zero_rows(zbuf, width // 16, 64)
    for r in range(RPS // 64):
        pltpu.sync_copy(zbuf, acc_sh.at[pl.ds(s * RPS + r * 64, 64)])


def _sc_deg_body(dst_hbm, out, onesb, zbuf, didx, acc_sh, isem):
    c = lax.axis_index("c")
    s = lax.axis_index("s")
    wch = (c * NS + s) * DEG_NCHUNK

    one = jnp.ones((16,), jnp.float32)

    def fill_ones(i, _):
        onesb[i, pl.ds(0, 16)] = one
        return 0

    lax.fori_loop(0, CH, fill_ones, 0, unroll=4)
    _zero_acc(zbuf, acc_sh, s, 16)
    plsc.subcore_barrier()

    def stage(gi, p):
        pltpu.async_copy(dst_hbm.at[pl.ds(wch + gi * NBUF, NBUF)], didx.at[p],
                         isem.at[p])

    def wait_stage(p):
        pltpu.make_async_copy(dst_hbm.at[pl.ds(0, NBUF)], didx.at[p],
                              isem.at[p]).wait()

    def scatters(p):
        for b in range(NBUF):
            pltpu.sync_copy(onesb, acc_sh.at[didx.at[p, b]], add=True)

    stage(0, 0)
    stage(1, 1)

    def go_body(go, _):
        g0 = 2 * go
        wait_stage(0)
        scatters(0)

        @pl.when(g0 + 2 < DEG_NGRP)
        def _():
            stage(g0 + 2, 0)

        wait_stage(1)
        scatters(1)

        @pl.when(g0 + 3 < DEG_NGRP)
        def _():
            stage(g0 + 3, 1)

        return 0

    lax.fori_loop(0, DEG_NGRP // 2, go_body, 0)
    plsc.subcore_barrier()
    pltpu.sync_copy(acc_sh.at[pl.ds(s * RPS, RPS)], out.at[c, pl.ds(s * RPS, RPS)])


_sc_deg = pl.kernel(
    _sc_deg_body,
    out_type=jax.ShapeDtypeStruct((NC, NPAD, 16), jnp.float32),
    mesh=_mesh,
    scratch_types=[
        pltpu.VMEM((CH, 16), jnp.float32),
        pltpu.VMEM((64, 16), jnp.float32),
        pltpu.VMEM((2, NBUF, CH), jnp.int32),
        pltpu.VMEM_SHARED((NPAD, 16), jnp.float32),
        pltpu.SemaphoreType.DMA((2,)),
    ],
    compiler_params=_sc_params,
)


def _sc_scatter_body(g2_hbm, src_hbm, dst_hbm, out,
                     sidx, didx, rows, zbuf, acc_sh, g_sh, ssem, dsem, gsem):
    c = lax.axis_index("c")
    s = lax.axis_index("s")
    wch = s * SCT_NCHUNK

    pltpu.sync_copy(g2_hbm.at[c, pl.ds(s * RPS, RPS)], g_sh.at[pl.ds(s * RPS, RPS)])
    _zero_acc(zbuf, acc_sh, s, DH)
    plsc.subcore_barrier()

    def stage_src(gi, p):
        pltpu.async_copy(src_hbm.at[pl.ds(wch + gi * NBUF, NBUF)], sidx.at[p],
                         ssem.at[p])

    def stage_dst(gi, p):
        pltpu.async_copy(dst_hbm.at[pl.ds(wch + gi * NBUF, NBUF)], didx.at[p],
                         dsem.at[p])

    def wait_src(p):
        pltpu.make_async_copy(src_hbm.at[pl.ds(0, NBUF)], sidx.at[p],
                              ssem.at[p]).wait()

    def wait_dst(p):
        pltpu.make_async_copy(dst_hbm.at[pl.ds(0, NBUF)], didx.at[p],
                              dsem.at[p]).wait()

    def fire(p):
        for b in range(NBUF):
            pltpu.async_copy(g_sh.at[sidx.at[p, b]], rows.at[p, b],
                             gsem.at[p, b])

    def wait_gathers(p):
        for b in range(NBUF):
            pltpu.make_async_copy(g_sh.at[sidx.at[p, b]], rows.at[p, b],
                                  gsem.at[p, b]).wait()

    def scatters(p):
        for b in range(NBUF):
            pltpu.sync_copy(rows.at[p, b], acc_sh.at[didx.at[p, b]], add=True)

    stage_src(0, 0)
    stage_dst(0, 0)
    wait_src(0)
    fire(0)
    stage_src(1, 1)
    stage_dst(1, 1)

    def go_body(go, _):
        g0 = 2 * go
        wait_src(1)
        fire(1)
        wait_gathers(0)

        @pl.when(g0 + 2 < SCT_NGRP)
        def _():
            stage_src(g0 + 2, 0)

        wait_dst(0)
        scatters(0)

        @pl.when(g0 + 2 < SCT_NGRP)
        def _():
            stage_dst(g0 + 2, 0)
            wait_src(0)
            fire(0)

        wait_gathers(1)

        @pl.when(g0 + 3 < SCT_NGRP)
        def _():
            stage_src(g0 + 3, 1)

        wait_dst(1)
        scatters(1)

        @pl.when(g0 + 3 < SCT_NGRP)
        def _():
            stage_dst(g0 + 3, 1)

        return 0

    lax.fori_loop(0, SCT_NGRP // 2, go_body, 0)
    plsc.subcore_barrier()
    pltpu.sync_copy(acc_sh.at[pl.ds(s * RPS, RPS)], out.at[c, pl.ds(s * RPS, RPS)])


_sc_scatter = pl.kernel(
    _sc_scatter_body,
    out_type=jax.ShapeDtypeStruct((NC, NPAD, DH), jnp.float32),
    mesh=_mesh,
    scratch_types=[
        pltpu.VMEM((2, NBUF, CH), jnp.int32),
        pltpu.VMEM((2, NBUF, CH), jnp.int32),
        pltpu.VMEM((2, NBUF, CH, DH), jnp.float32),
        pltpu.VMEM((64, DH), jnp.float32),
        pltpu.VMEM_SHARED((NPAD, DH), jnp.float32),
        pltpu.VMEM_SHARED((NPAD, DH), jnp.float32),
        pltpu.SemaphoreType.DMA((2,)),
        pltpu.SemaphoreType.DMA((2,)),
        pltpu.SemaphoreType.DMA((2, NBUF)),
    ],
    compiler_params=_sc_params,
)


TCR = 2048
TCG = NPAD // TCR


def _dinv(d0r, d1r):
    deg = d0r[:, :1] + d1r[:, :1] + 1.0
    return lax.rsqrt(jnp.maximum(deg, 1.0))


def _split_g(gr, v):
    gr[0] = v[:, :DH]
    gr[1] = v[:, DH:]


def _tc1_body(xr, wr, d0r, d1r, h1r, g1r):
    dinv = _dinv(d0r, d1r)
    h = jnp.dot(xr[:], wr[:], preferred_element_type=jnp.float32,
                precision=lax.Precision.HIGHEST)
    h1r[:] = h
    _split_g(g1r, h * dinv)


def _tc2_body(p0r, p1r, d0r, d1r, h1r, b1r, w2r, h2r, g2r):
    i = pl.program_id(0)
    dinv = _dinv(d0r, d1r)
    acc = jnp.concatenate([p0r[:], p1r[:]], axis=1)
    z = jnp.maximum(dinv * acc + (dinv * dinv) * h1r[:] + b1r[:], 0.0)
    rows = i * TCR + lax.broadcasted_iota(jnp.int32, (TCR, 1), 0)
    z = jnp.where(rows < N, z, 0.0)
    h2 = jnp.dot(z, w2r[:], preferred_element_type=jnp.float32,
                 precision=lax.Precision.HIGHEST)
    h2r[:] = h2
    _split_g(g2r, h2 * dinv)


def _tc3_body(p0r, p1r, d0r, d1r, h2r, b2r, wlr, blr, outr):
    dinv = _dinv(d0r, d1r)
    acc = jnp.concatenate([p0r[:], p1r[:]], axis=1)
    z = jnp.maximum(dinv * acc + (dinv * dinv) * h2r[:] + b2r[:], 0.0)
    outr[:] = jnp.sum(z * wlr[:], axis=1, keepdims=True) + blr[:]


def _row_spec(w):
    return pl.BlockSpec((TCR, w), lambda i: (i, 0))


def _full_spec(a, b):
    return pl.BlockSpec((a, b), lambda i: (0, 0))


_gsplit_spec = pl.BlockSpec((2, TCR, DH), lambda i: (0, i, 0))
_gsplit_shape = jax.ShapeDtypeStruct((2, NPAD, DH), jnp.float32)

_tc1 = pl.pallas_call(
    _tc1_body,
    grid=(TCG,),
    in_specs=[_row_spec(IN_CH), _full_spec(IN_CH, D), _row_spec(16), _row_spec(16)],
    out_specs=[_row_spec(D), _gsplit_spec],
    out_shape=[jax.ShapeDtypeStruct((NPAD, D), jnp.float32), _gsplit_shape],
)

_tc2 = pl.pallas_call(
    _tc2_body,
    grid=(TCG,),
    in_specs=[_row_spec(DH), _row_spec(DH), _row_spec(16), _row_spec(16),
              _row_spec(D), _full_spec(1, D), _full_spec(D, D)],
    out_specs=[_row_spec(D), _gsplit_spec],
    out_shape=[jax.ShapeDtypeStruct((NPAD, D), jnp.float32), _gsplit_shape],
)

_tc3 = pl.pallas_call(
    _tc3_body,
    grid=(TCG,),
    in_specs=[_row_spec(DH), _row_spec(DH), _row_spec(16), _row_spec(16),
              _row_spec(D), _full_spec(1, D), _full_spec(1, D), _full_spec(1, 1)],
    out_specs=[pl.BlockSpec((TCR, 1), lambda i: (i, 0))],
    out_shape=[jax.ShapeDtypeStruct((NPAD, 1), jnp.float32)],
)


@jax.jit
def kernel(x, edge_index, W1, b1, W2, b2, Wlin, blin):
    ei = edge_index.astype(jnp.int32)
    pad = EPAD - E
    trash = N + jnp.arange(pad, dtype=jnp.int32) % (NPAD - N)
    src = jnp.concatenate([ei[0], jnp.full((pad,), N, jnp.int32)]).reshape(TOTCH, CH)
    dst = jnp.concatenate([ei[1], trash]).reshape(TOTCH, CH)
    x_pad = jnp.pad(x, ((0, NPAD - N), (0, 0)))
    b1r = b1.reshape(1, D)
    b2r = b2.reshape(1, D)
    wlr = Wlin.reshape(1, D)
    blr = blin.reshape(1, 1)

    dd = _sc_deg(dst)
    d0, d1 = dd[0], dd[1]
    h1, g1 = _tc1(x_pad, W1, d0, d1)
    pp = _sc_scatter(g1, src, dst)
    h2, g2 = _tc2(pp[0], pp[1], d0, d1, h1, b1r, W2)
    qq = _sc_scatter(g2, src, dst)
    out = _tc3(qq[0], qq[1], d0, d1, h2, b2r, wlr, blr)[0]
    return out[:N, 0]


# speedup vs baseline: 33.0332x; 1.0244x over previous
import functools

import jax
import jax.numpy as jnp
from jax import lax
from jax.experimental import pallas as pl
from jax.experimental.pallas import tpu as pltpu
from jax.experimental.pallas import tpu_sc as plsc

N = 10000
NPAD = 10240
E = 320000
IN_CH = 128
D = 64
DH = D // 2

NC = 2
NS = 16
NW = NC * NS
CH = 128
NBUF = 8
TOTCH = 2560
EPAD = TOTCH * CH
RPS = NPAD // NS

DEG_NCHUNK = TOTCH // NW
DEG_NGRP = DEG_NCHUNK // NBUF
SCT_NCHUNK = TOTCH // NS
SCT_NGRP = SCT_NCHUNK // NBUF

_mesh = plsc.VectorSubcoreMesh(
    core_axis_name="c", subcore_axis_name="s", num_cores=NC, num_subcores=NS
)
_sc_params = pltpu.CompilerParams(use_tc_tiling_on_sc=False)


def _zero_rows(zbuf, width_vregs, nrows):
    zero = jnp.zeros((16,), jnp.float32)

    def body(i, _):
        for j in range(width_vregs):
            zbuf[i, pl.ds(j * 16, 16)] = zero
        return 0

    lax.fori_loop(0, nrows, body, 0, unroll=4)


def _zero_acc(zbuf, acc_sh, s, width):
    _zero_rows(zbuf, width // 16, 64)
    for r in range(RPS // 64):
        pltpu.sync_copy(zbuf, acc_sh.at[pl.ds(s * RPS + r * 64, 64)])


def _sc_deg_body(dst_hbm, out, onesb, zbuf, didx, acc_sh, isem, csem):
    c = lax.axis_index("c")
    s = lax.axis_index("s")
    wch = (c * NS + s) * DEG_NCHUNK

    one = jnp.ones((16,), jnp.float32)

    def fill_ones(i, _):
        onesb[i, pl.ds(0, 16)] = one
        return 0

    lax.fori_loop(0, CH, fill_ones, 0, unroll=4)
    _zero_acc(zbuf, acc_sh, s, 16)
    plsc.subcore_barrier()

    def stage(gi, p):
        pltpu.async_copy(dst_hbm.at[pl.ds(wch + gi * NBUF, NBUF)], didx.at[p],
                         isem.at[p])

    def wait_stage(p):
        pltpu.make_async_copy(dst_hbm.at[pl.ds(0, NBUF)], didx.at[p],
                              isem.at[p]).wait()

    def scatters(p):
        for b in range(NBUF):
            pltpu.async_copy(onesb, acc_sh.at[didx.at[p, b]], csem.at[p],
                             add=True)

    def drain_scatters(p):
        for b in range(NBUF):
            pltpu.make_async_copy(onesb, acc_sh.at[didx.at[p, b]],
                                  csem.at[p]).wait()

    stage(0, 0)
    stage(1, 1)

    def go_body(go, _):
        g0 = 2 * go
        wait_stage(0)
        scatters(0)
        drain_scatters(0)

        @pl.when(g0 + 2 < DEG_NGRP)
        def _():
            stage(g0 + 2, 0)

        wait_stage(1)
        scatters(1)
        drain_scatters(1)

        @pl.when(g0 + 3 < DEG_NGRP)
        def _():
            stage(g0 + 3, 1)

        return 0

    lax.fori_loop(0, DEG_NGRP // 2, go_body, 0)
    plsc.subcore_barrier()
    pltpu.sync_copy(acc_sh.at[pl.ds(s * RPS, RPS)], out.at[c, pl.ds(s * RPS, RPS)])


_sc_deg = pl.kernel(
    _sc_deg_body,
    out_type=jax.ShapeDtypeStruct((NC, NPAD, 16), jnp.float32),
    mesh=_mesh,
    scratch_types=[
        pltpu.VMEM((CH, 16), jnp.float32),
        pltpu.VMEM((64, 16), jnp.float32),
        pltpu.VMEM((2, NBUF, CH), jnp.int32),
        pltpu.VMEM_SHARED((NPAD, 16), jnp.float32),
        pltpu.SemaphoreType.DMA((2,)),
        pltpu.SemaphoreType.DMA((2,)),
    ],
    compiler_params=_sc_params,
)


def _sc_scatter_body(g2_hbm, src_hbm, dst_hbm, out, sidx, didx, rows, zbuf,
                     acc_sh, g_sh, ssem, dsem, gsem, csem):
    c = lax.axis_index("c")
    s = lax.axis_index("s")
    wch = s * SCT_NCHUNK

    pltpu.sync_copy(g2_hbm.at[c, pl.ds(s * RPS, RPS)], g_sh.at[pl.ds(s * RPS, RPS)])
    _zero_acc(zbuf, acc_sh, s, DH)
    plsc.subcore_barrier()

    def stage_src(gi, p):
        pltpu.async_copy(src_hbm.at[pl.ds(wch + gi * NBUF, NBUF)], sidx.at[p],
                         ssem.at[p])

    def stage_dst(gi, p):
        pltpu.async_copy(dst_hbm.at[pl.ds(wch + gi * NBUF, NBUF)], didx.at[p],
                         dsem.at[p])

    def wait_src(p):
        pltpu.make_async_copy(src_hbm.at[pl.ds(0, NBUF)], sidx.at[p],
                              ssem.at[p]).wait()

    def wait_dst(p):
        pltpu.make_async_copy(dst_hbm.at[pl.ds(0, NBUF)], didx.at[p],
                              dsem.at[p]).wait()

    def fire(p):
        for b in range(NBUF):
            pltpu.async_copy(g_sh.at[sidx.at[p, b]], rows.at[p, b],
                             gsem.at[p, b])

    def wait_gathers(p):
        for b in range(NBUF):
            pltpu.make_async_copy(g_sh.at[sidx.at[p, b]], rows.at[p, b],
                                  gsem.at[p, b]).wait()

    def scatters(p):
        for b in range(NBUF):
            pltpu.async_copy(rows.at[p, b], acc_sh.at[didx.at[p, b]],
                             csem.at[p], add=True)

    def drain_scatters(p):
        for b in range(NBUF):
            pltpu.make_async_copy(rows.at[p, b], acc_sh.at[didx.at[p, b]],
                                  csem.at[p]).wait()

    stage_src(0, 0)
    stage_dst(0, 0)
    wait_src(0)
    fire(0)
    stage_src(1, 1)
    stage_dst(1, 1)

    def go_body(go, _):
        g0 = 2 * go
        wait_src(1)
        fire(1)
        wait_gathers(0)

        @pl.when(g0 + 2 < SCT_NGRP)
        def _():
            stage_src(g0 + 2, 0)

        wait_dst(0)
        scatters(0)
        wait_gathers(1)

        @pl.when(g0 + 3 < SCT_NGRP)
        def _():
            stage_src(g0 + 3, 1)

        drain_scatters(0)

        @pl.when(g0 + 2 < SCT_NGRP)
        def _():
            stage_dst(g0 + 2, 0)
            wait_src(0)
            fire(0)

        wait_dst(1)
        scatters(1)
        drain_scatters(1)

        @pl.when(g0 + 3 < SCT_NGRP)
        def _():
            stage_dst(g0 + 3, 1)

        return 0

    lax.fori_loop(0, SCT_NGRP // 2, go_body, 0)
    plsc.subcore_barrier()
    pltpu.sync_copy(acc_sh.at[pl.ds(s * RPS, RPS)], out.at[c, pl.ds(s * RPS, RPS)])


_sc_scatter = pl.kernel(
    _sc_scatter_body,
    out_type=jax.ShapeDtypeStruct((NC, NPAD, DH), jnp.float32),
    mesh=_mesh,
    scratch_types=[
        pltpu.VMEM((2, NBUF, CH), jnp.int32),
        pltpu.VMEM((2, NBUF, CH), jnp.int32),
        pltpu.VMEM((2, NBUF, CH, DH), jnp.float32),
        pltpu.VMEM((64, DH), jnp.float32),
        pltpu.VMEM_SHARED((NPAD, DH), jnp.float32),
        pltpu.VMEM_SHARED((NPAD, DH), jnp.float32),
        pltpu.SemaphoreType.DMA((2,)),
        pltpu.SemaphoreType.DMA((2,)),
        pltpu.SemaphoreType.DMA((2, NBUF)),
        pltpu.SemaphoreType.DMA((2,)),
    ],
    compiler_params=_sc_params,
)


TCR = 2048
TCG = NPAD // TCR


def _dinv(d0r, d1r):
    deg = d0r[:, :1] + d1r[:, :1] + 1.0
    return lax.rsqrt(jnp.maximum(deg, 1.0))


def _split_g(gr, v):
    gr[0] = v[:, :DH]
    gr[1] = v[:, DH:]


def _tc1_body(xr, wr, d0r, d1r, h1r, g1r):
    dinv = _dinv(d0r, d1r)
    h = jnp.dot(xr[:], wr[:], preferred_element_type=jnp.float32,
                precision=lax.Precision.HIGHEST)
    h1r[:] = h
    _split_g(g1r, h * dinv)


def _tc2_body(p0r, p1r, d0r, d1r, h1r, b1r, w2r, h2r, g2r):
    i = pl.program_id(0)
    dinv = _dinv(d0r, d1r)
    acc = jnp.concatenate([p0r[:], p1r[:]], axis=1)
    z = jnp.maximum(dinv * acc + (dinv * dinv) * h1r[:] + b1r[:], 0.0)
    rows = i * TCR + lax.broadcasted_iota(jnp.int32, (TCR, 1), 0)
    z = jnp.where(rows < N, z, 0.0)
    h2 = jnp.dot(z, w2r[:], preferred_element_type=jnp.float32,
                 precision=lax.Precision.HIGHEST)
    h2r[:] = h2
    _split_g(g2r, h2 * dinv)


def _tc3_body(p0r, p1r, d0r, d1r, h2r, b2r, wlr, blr, outr):
    dinv = _dinv(d0r, d1r)
    acc = jnp.concatenate([p0r[:], p1r[:]], axis=1)
    z = jnp.maximum(dinv * acc + (dinv * dinv) * h2r[:] + b2r[:], 0.0)
    outr[:] = jnp.sum(z * wlr[:], axis=1, keepdims=True) + blr[:]


def _row_spec(w):
    return pl.BlockSpec((TCR, w), lambda i: (i, 0))


def _full_spec(a, b):
    return pl.BlockSpec((a, b), lambda i: (0, 0))


_gsplit_spec = pl.BlockSpec((2, TCR, DH), lambda i: (0, i, 0))
_gsplit_shape = jax.ShapeDtypeStruct((2, NPAD, DH), jnp.float32)

_tc1 = pl.pallas_call(
    _tc1_body,
    grid=(TCG,),
    in_specs=[_row_spec(IN_CH), _full_spec(IN_CH, D), _row_spec(16), _row_spec(16)],
    out_specs=[_row_spec(D), _gsplit_spec],
    out_shape=[jax.ShapeDtypeStruct((NPAD, D), jnp.float32), _gsplit_shape],
)

_tc2 = pl.pallas_call(
    _tc2_body,
    grid=(TCG,),
    in_specs=[_row_spec(DH), _row_spec(DH), _row_spec(16), _row_spec(16),
              _row_spec(D), _full_spec(1, D), _full_spec(D, D)],
    out_specs=[_row_spec(D), _gsplit_spec],
    out_shape=[jax.ShapeDtypeStruct((NPAD, D), jnp.float32), _gsplit_shape],
)

_tc3 = pl.pallas_call(
    _tc3_body,
    grid=(TCG,),
    in_specs=[_row_spec(DH), _row_spec(DH), _row_spec(16), _row_spec(16),
              _row_spec(D), _full_spec(1, D), _full_spec(1, D), _full_spec(1, 1)],
    out_specs=[pl.BlockSpec((TCR, 1), lambda i: (i, 0))],
    out_shape=[jax.ShapeDtypeStruct((NPAD, 1), jnp.float32)],
)


@jax.jit
def kernel(x, edge_index, W1, b1, W2, b2, Wlin, blin):
    ei = edge_index.astype(jnp.int32)
    pad = EPAD - E
    trash = N + jnp.arange(pad, dtype=jnp.int32) % (NPAD - N)
    src = jnp.concatenate([ei[0], jnp.full((pad,), N, jnp.int32)]).reshape(TOTCH, CH)
    dst = jnp.concatenate([ei[1], trash]).reshape(TOTCH, CH)
    x_pad = jnp.pad(x, ((0, NPAD - N), (0, 0)))
    b1r = b1.reshape(1, D)
    b2r = b2.reshape(1, D)
    wlr = Wlin.reshape(1, D)
    blr = blin.reshape(1, 1)

    dd = _sc_deg(dst)
    d0, d1 = dd[0], dd[1]
    h1, g1 = _tc1(x_pad, W1, d0, d1)
    pp = _sc_scatter(g1, src, dst)
    h2, g2 = _tc2(pp[0], pp[1], d0, d1, h1, b1r, W2)
    qq = _sc_scatter(g2, src, dst)
    out = _tc3(qq[0], qq[1], d0, d1, h2, b2r, wlr, blr)[0]
    return out[:N, 0]


# speedup vs baseline: 36.3475x; 1.1003x over previous
import functools

import jax
import jax.numpy as jnp
from jax import lax
from jax.experimental import pallas as pl
from jax.experimental.pallas import tpu as pltpu
from jax.experimental.pallas import tpu_sc as plsc

N = 10000
NPAD = 10240
E = 320000
IN_CH = 128
D = 64
DH = D // 2

NC = 2
NS = 16
NW = NC * NS
CH = 128
NBUF = 8
TOTCH = 2560
EPAD = TOTCH * CH
RPS = NPAD // NS

DEG_NCHUNK = TOTCH // NW
DEG_NGRP = DEG_NCHUNK // NBUF
SCT_NCHUNK = TOTCH // NS
SCT_NGRP = SCT_NCHUNK // NBUF

_mesh = plsc.VectorSubcoreMesh(
    core_axis_name="c", subcore_axis_name="s", num_cores=NC, num_subcores=NS
)
_sc_params = pltpu.CompilerParams(use_tc_tiling_on_sc=False)


def _zero_rows(zbuf, width_vregs, nrows):
    zero = jnp.zeros((16,), jnp.float32)

    def body(i, _):
        for j in range(width_vregs):
            zbuf[i, pl.ds(j * 16, 16)] = zero
        return 0

    lax.fori_loop(0, nrows, body, 0, unroll=4)


def _zero_acc(zbuf, acc_sh, s, width):
    _zero_rows(zbuf, width // 16, 64)
    for r in range(RPS // 64):
        pltpu.sync_copy(zbuf, acc_sh.at[pl.ds(s * RPS + r * 64, 64)])


def _sc_deg_body(dst_hbm, out, onesb, zbuf, didx, acc_sh, isem, csem):
    c = lax.axis_index("c")
    s = lax.axis_index("s")
    wch = (c * NS + s) * DEG_NCHUNK

    one = jnp.ones((16,), jnp.float32)

    def fill_ones(i, _):
        onesb[i, pl.ds(0, 16)] = one
        return 0

    lax.fori_loop(0, CH, fill_ones, 0, unroll=4)
    _zero_acc(zbuf, acc_sh, s, 16)
    plsc.subcore_barrier()

    def stage(gi, p):
        pltpu.async_copy(dst_hbm.at[pl.ds(wch + gi * NBUF, NBUF)], didx.at[p],
                         isem.at[p])

    def wait_stage(p):
        pltpu.make_async_copy(dst_hbm.at[pl.ds(0, NBUF)], didx.at[p],
                              isem.at[p]).wait()

    def scatters(p):
        for b in range(NBUF):
            pltpu.async_copy(onesb, acc_sh.at[didx.at[p, b]], csem.at[p],
                             add=True)

    def drain_scatters(p):
        for b in range(NBUF):
            pltpu.make_async_copy(onesb, acc_sh.at[didx.at[p, b]],
                                  csem.at[p]).wait()

    stage(0, 0)
    stage(1, 1)

    def go_body(go, _):
        g0 = 2 * go
        wait_stage(0)
        scatters(0)
        drain_scatters(0)

        @pl.when(g0 + 2 < DEG_NGRP)
        def _():
            stage(g0 + 2, 0)

        wait_stage(1)
        scatters(1)
        drain_scatters(1)

        @pl.when(g0 + 3 < DEG_NGRP)
        def _():
            stage(g0 + 3, 1)

        return 0

    lax.fori_loop(0, DEG_NGRP // 2, go_body, 0)
    plsc.subcore_barrier()
    pltpu.sync_copy(acc_sh.at[pl.ds(s * RPS, RPS)], out.at[c, pl.ds(s * RPS, RPS)])


_sc_deg = pl.kernel(
    _sc_deg_body,
    out_type=jax.ShapeDtypeStruct((NC, NPAD, 16), jnp.float32),
    mesh=_mesh,
    scratch_types=[
        pltpu.VMEM((CH, 16), jnp.float32),
        pltpu.VMEM((64, 16), jnp.float32),
        pltpu.VMEM((2, NBUF, CH), jnp.int32),
        pltpu.VMEM_SHARED((NPAD, 16), jnp.float32),
        pltpu.SemaphoreType.DMA((2,)),
        pltpu.SemaphoreType.DMA((2,)),
    ],
    compiler_params=_sc_params,
)


def _sc_scatter_body(g2_hbm, src_hbm, dst_hbm, out, sidx, didx, rows, zbuf,
                     acc_sh, g_sh, ssem, dsem, gsem, csem):
    c = lax.axis_index("c")
    s = lax.axis_index("s")
    wch = s * SCT_NCHUNK

    pltpu.sync_copy(g2_hbm.at[c, pl.ds(s * RPS, RPS)], g_sh.at[pl.ds(s * RPS, RPS)])
    _zero_acc(zbuf, acc_sh, s, DH)
    plsc.subcore_barrier()

    def stage_src(gi, p):
        pltpu.async_copy(src_hbm.at[pl.ds(wch + gi * NBUF, NBUF)], sidx.at[p],
                         ssem.at[p])

    def stage_dst(gi, p):
        pltpu.async_copy(dst_hbm.at[pl.ds(wch + gi * NBUF, NBUF)], didx.at[p],
                         dsem.at[p])

    def wait_src(p):
        pltpu.make_async_copy(src_hbm.at[pl.ds(0, NBUF)], sidx.at[p],
                              ssem.at[p]).wait()

    def wait_dst(p):
        pltpu.make_async_copy(dst_hbm.at[pl.ds(0, NBUF)], didx.at[p],
                              dsem.at[p]).wait()

    def fire(p):
        for b in range(NBUF):
            pltpu.async_copy(g_sh.at[sidx.at[p, b]], rows.at[p, b],
                             gsem.at[p, b])

    def wait_gathers(p):
        for b in range(NBUF):
            pltpu.make_async_copy(g_sh.at[sidx.at[p, b]], rows.at[p, b],
                                  gsem.at[p, b]).wait()

    def scatters(p):
        for b in range(NBUF):
            pltpu.async_copy(rows.at[p, b], acc_sh.at[didx.at[p, b]],
                             csem.at[p], add=True)

    def drain_scatters(p):
        for b in range(NBUF):
            pltpu.make_async_copy(rows.at[p, b], acc_sh.at[didx.at[p, b]],
                                  csem.at[p]).wait()

    stage_src(0, 0)
    stage_dst(0, 0)
    wait_src(0)
    fire(0)
    stage_src(1, 1)
    stage_dst(1, 1)

    def go_body(go, _):
        g0 = 2 * go
        wait_src(1)
        fire(1)
        wait_gathers(0)

        @pl.when(g0 + 2 < SCT_NGRP)
        def _():
            stage_src(g0 + 2, 0)

        wait_dst(0)
        scatters(0)
        wait_gathers(1)

        @pl.when(g0 + 3 < SCT_NGRP)
        def _():
            stage_src(g0 + 3, 1)

        drain_scatters(0)

        @pl.when(g0 + 2 < SCT_NGRP)
        def _():
            stage_dst(g0 + 2, 0)
            wait_src(0)
            fire(0)

        wait_dst(1)
        scatters(1)
        drain_scatters(1)

        @pl.when(g0 + 3 < SCT_NGRP)
        def _():
            stage_dst(g0 + 3, 1)

        return 0

    lax.fori_loop(0, SCT_NGRP // 2, go_body, 0)
    plsc.subcore_barrier()
    pltpu.sync_copy(acc_sh.at[pl.ds(s * RPS, RPS)], out.at[c, pl.ds(s * RPS, RPS)])


_sc_scatter = pl.kernel(
    _sc_scatter_body,
    out_type=jax.ShapeDtypeStruct((NC, NPAD, DH), jnp.float32),
    mesh=_mesh,
    scratch_types=[
        pltpu.VMEM((2, NBUF, CH), jnp.int32),
        pltpu.VMEM((2, NBUF, CH), jnp.int32),
        pltpu.VMEM((2, NBUF, CH, DH), jnp.float32),
        pltpu.VMEM((64, DH), jnp.float32),
        pltpu.VMEM_SHARED((NPAD, DH), jnp.float32),
        pltpu.VMEM_SHARED((NPAD, DH), jnp.float32),
        pltpu.SemaphoreType.DMA((2,)),
        pltpu.SemaphoreType.DMA((2,)),
        pltpu.SemaphoreType.DMA((2, NBUF)),
        pltpu.SemaphoreType.DMA((2,)),
    ],
    compiler_params=_sc_params,
)


TCR = 2560
TCG = NPAD // TCR


def _dinv(ddr):
    deg = ddr[0, :, :1] + ddr[1, :, :1] + 1.0
    return lax.rsqrt(jnp.maximum(deg, 1.0))


def _split_g(gr, v):
    gr[0] = v[:, :DH]
    gr[1] = v[:, DH:]


def _tc1a_body(xr, wr, h1r):
    h1r[:] = jnp.dot(xr[:], wr[:], preferred_element_type=jnp.float32,
                     precision=lax.Precision.HIGHEST)


def _tc1b_body(h1r, ddr, g1r):
    _split_g(g1r, h1r[:] * _dinv(ddr))


def _tc2_body(ppr, ddr, h1r, b1r, w2r, h2r, g2r):
    i = pl.program_id(0)
    dinv = _dinv(ddr)
    acc = jnp.concatenate([ppr[0], ppr[1]], axis=1)
    z = jnp.maximum(dinv * acc + (dinv * dinv) * h1r[:] + b1r[:], 0.0)
    rows = i * TCR + lax.broadcasted_iota(jnp.int32, (TCR, 1), 0)
    z = jnp.where(rows < N, z, 0.0)
    h2 = jnp.dot(z, w2r[:], preferred_element_type=jnp.float32,
                 precision=lax.Precision.HIGHEST)
    h2r[:] = h2
    _split_g(g2r, h2 * dinv)


def _tc3_body(qqr, ddr, h2r, b2r, wlr, blr, outr):
    dinv = _dinv(ddr)
    acc = jnp.concatenate([qqr[0], qqr[1]], axis=1)
    z = jnp.maximum(dinv * acc + (dinv * dinv) * h2r[:] + b2r[:], 0.0)
    outr[:] = jnp.sum(z * wlr[:], axis=1, keepdims=True) + blr[:]


def _rows(w):
    return pl.BlockSpec((TCR, w), lambda i: (i, 0))


def _full(a, b):
    return pl.BlockSpec((a, b), lambda i: (0, 0))


def _rows3(w):
    return pl.BlockSpec((2, TCR, w), lambda i: (0, i, 0))


_gsplit_shape = jax.ShapeDtypeStruct((2, NPAD, DH), jnp.float32)
_h_shape = jax.ShapeDtypeStruct((NPAD, D), jnp.float32)

_tc1a = pl.pallas_call(
    _tc1a_body, grid=(TCG,),
    in_specs=[_rows(IN_CH), _full(IN_CH, D)],
    out_specs=_rows(D), out_shape=_h_shape)
_tc1b = pl.pallas_call(
    _tc1b_body, grid=(TCG,),
    in_specs=[_rows(D), _rows3(16)],
    out_specs=_rows3(DH), out_shape=_gsplit_shape)
_tc2 = pl.pallas_call(
    _tc2_body, grid=(TCG,),
    in_specs=[_rows3(DH), _rows3(16), _rows(D), _full(1, D), _full(D, D)],
    out_specs=[_rows(D), _rows3(DH)],
    out_shape=[_h_shape, _gsplit_shape])
_tc3 = pl.pallas_call(
    _tc3_body, grid=(TCG,),
    in_specs=[_rows3(DH), _rows3(16), _rows(D), _full(1, D), _full(1, D),
              _full(1, 1)],
    out_specs=pl.BlockSpec((TCR, 1), lambda i: (i, 0)),
    out_shape=jax.ShapeDtypeStruct((NPAD, 1), jnp.float32))


@jax.jit
def kernel(x, edge_index, W1, b1, W2, b2, Wlin, blin):
    ei = edge_index.astype(jnp.int32)
    pad = EPAD - E
    trash = N + jnp.arange(pad, dtype=jnp.int32) % (NPAD - N)
    src = jnp.concatenate([ei[0], jnp.full((pad,), N, jnp.int32)]).reshape(TOTCH, CH)
    dst = jnp.concatenate([ei[1], trash]).reshape(TOTCH, CH)
    x_pad = jnp.pad(x, ((0, NPAD - N), (0, 0)))
    b1r = b1.reshape(1, D)
    b2r = b2.reshape(1, D)
    wlr = Wlin.reshape(1, D)
    blr = blin.reshape(1, 1)

    dd = _sc_deg(dst)
    h1 = _tc1a(x_pad, W1)
    g1 = _tc1b(h1, dd)
    pp = _sc_scatter(g1, src, dst)
    h2, g2 = _tc2(pp, dd, h1, b1r, W2)
    qq = _sc_scatter(g2, src, dst)
    out = _tc3(qq, dd, h2, b2r, wlr, blr)
    return out[:N, 0]
